# baseline (speedup 1.0000x reference)
import functools

import jax
import jax.numpy as jnp
from jax import lax
from jax.experimental import pallas as pl
from jax.experimental.pallas import tpu as pltpu
from jax.experimental.pallas import tpu_sc as plsc

BATCH = 16384
FIELDS = 26
EMBED = 32
TOTAL = BATCH * FIELDS

NC = 2
NS = 16
NW = NC * NS
NTB = BATCH // 128
TB_PER_W = NTB // NW
PER_W = TB_PER_W * 128 * FIELDS
BLOCKS_PER_W = FIELDS * TB_PER_W
CHUNK = 8
NCHUNKS = BLOCKS_PER_W // CHUNK
TPAD = 133


def _emb_body(idx_hbm, tab_hbm, o_hbm, idx_v, gl, rows, tiles, gsem, wsem):
    wid = lax.axis_index("s") * NC + lax.axis_index("c")
    tb0 = wid * TB_PER_W
    base = wid * PER_W
    pltpu.sync_copy(idx_hbm.at[pl.ds(base, PER_W)], idx_v)

    iota = lax.broadcasted_iota(jnp.int32, (16,), 0)
    iota26 = iota * FIELDS
    te_vec = iota // 8
    es_vec = iota % 8

    def fire(blk, k):
        f = blk % FIELDS
        tbl = blk // FIELDS
        gbase = tbl * (128 * FIELDS) + f
        vs = [
            plsc.load_gather(idx_v, [gbase + g * (16 * FIELDS) + iota26])
            for g in range(8)
        ]
        for g in range(8):
            gl[k, pl.ds(g * 16, 16)] = vs[g]
        return pltpu.async_copy(tab_hbm.at[gl.at[k]], rows.at[k], gsem.at[k])

    def drain(blk, k, ghandle):
        f = blk % FIELDS
        tbl = blk // FIELDS
        par = k
        ghandle.wait()
        pltpu.make_async_copy(
            tiles.at[par, :, :, pl.ds(0, 128)],
            o_hbm.at[0, :, 0, :, :], wsem.at[par]).wait()
        tref = tiles.at[par]
        for r0 in range(0, 128, 8):
            vecs = [
                rows[k, r0 + j, pl.ds(h * 16, 16)]
                for j in range(8)
                for h in range(2)
            ]
            i = 0
            for j in range(8):
                for h in range(2):
                    plsc.store_scatter(
                        tref,
                        [te_vec + 2 * h, es_vec, jnp.full((16,), r0 + j,
                                                          jnp.int32)],
                        vecs[i])
                    i += 1
        return pltpu.async_copy(
            tiles.at[par, :, :, pl.ds(0, 128)],
            o_hbm.at[f, :, tb0 + tbl, :, :], wsem.at[par])

    for par in range(CHUNK):
        pltpu.async_copy(tiles.at[par, :, :, pl.ds(0, 128)],
                         o_hbm.at[par % 2, :, tb0, :, :], wsem.at[par])

    def chunk_body(ci, carry):
        blk0 = ci * CHUNK
        handles = [fire(blk0 + k, k) for k in range(CHUNK)]
        for k in range(CHUNK):
            drain(blk0 + k, k, handles[k])
        return carry

    lax.fori_loop(0, NCHUNKS, chunk_body, 0)
    for par in range(CHUNK):
        pltpu.make_async_copy(
            tiles.at[par, :, :, pl.ds(0, 128)],
            o_hbm.at[0, :, 0, :, :], wsem.at[par]).wait()


@jax.jit
def _emb(idx_flat, weight):
    mesh = plsc.VectorSubcoreMesh(core_axis_name="c", subcore_axis_name="s")
    run = pl.kernel(
        _emb_body,
        out_type=jax.ShapeDtypeStruct((FIELDS, 4, NTB, 8, 128), jnp.float32),
        mesh=mesh,
        scratch_types=[
            pltpu.VMEM((PER_W,), jnp.int32),
            pltpu.VMEM((CHUNK, 128), jnp.int32),
            pltpu.VMEM((CHUNK, 128, EMBED), jnp.float32),
            pltpu.VMEM((CHUNK, 4, 8, TPAD), jnp.float32),
            pltpu.SemaphoreType.DMA((CHUNK,)),
            pltpu.SemaphoreType.DMA((CHUNK,)),
        ],
        compiler_params=pltpu.CompilerParams(
            use_tc_tiling_on_sc=False, needs_layout_passes=False),
    )
    return run(idx_flat, weight)


def kernel(input, weight):
    idx_flat = input.reshape(TOTAL).astype(jnp.int32)
    out = _emb(idx_flat, weight)
    return out.transpose(2, 4, 0, 1, 3).reshape(BATCH, FIELDS, EMBED)


# speedup vs baseline: 1.0370x; 1.0370x over previous
import functools

import jax
import jax.numpy as jnp
from jax import lax
from jax.experimental import pallas as pl
from jax.experimental.pallas import tpu as pltpu
from jax.experimental.pallas import tpu_sc as plsc

BATCH = 16384
FIELDS = 26
EMBED = 32
TOTAL = BATCH * FIELDS

NC = 2
NS = 16
NW = NC * NS
NTB = BATCH // 128
TB_PER_W = NTB // NW
PER_W = TB_PER_W * 128 * FIELDS
BLOCKS_PER_W = FIELDS * TB_PER_W
CHUNK = 8
NCHUNKS = BLOCKS_PER_W // CHUNK
TPAD = 133


def _emb_body(idx_hbm, tab_hbm, o_hbm, idx_v, gl, rows, tiles, gsem, wsem):
    wid = lax.axis_index("s") * NC + lax.axis_index("c")
    tb0 = wid * TB_PER_W
    base = wid * PER_W
    pltpu.sync_copy(idx_hbm.at[pl.ds(base, PER_W)], idx_v)

    iota = lax.broadcasted_iota(jnp.int32, (16,), 0)
    iota26 = iota * FIELDS
    te_vec = iota // 8
    es_vec = iota % 8

    def fire(blk, k):
        f = blk % FIELDS
        tbl = blk // FIELDS
        gbase = tbl * (128 * FIELDS) + f
        vs = [
            plsc.load_gather(idx_v, [gbase + g * (16 * FIELDS) + iota26])
            for g in range(8)
        ]
        for g in range(8):
            gl[k, pl.ds(g * 16, 16)] = vs[g]
        pltpu.async_copy(tab_hbm.at[gl.at[k]], rows.at[k], gsem.at[k])

    def drain(blk, k):
        f = blk % FIELDS
        tbl = blk // FIELDS
        pltpu.make_async_copy(
            tab_hbm.at[gl.at[k]], rows.at[k], gsem.at[k]).wait()
        pltpu.make_async_copy(
            tiles.at[k, :, :, pl.ds(0, 128)],
            o_hbm.at[0, :, 0, :, :], wsem.at[k]).wait()
        tref = tiles.at[k]
        for r0 in range(0, 128, 8):
            vecs = [
                rows[k, r0 + j, pl.ds(h * 16, 16)]
                for j in range(8)
                for h in range(2)
            ]
            i = 0
            for j in range(8):
                for h in range(2):
                    plsc.store_scatter(
                        tref,
                        [te_vec + 2 * h, es_vec, jnp.full((16,), r0 + j,
                                                          jnp.int32)],
                        vecs[i])
                    i += 1
        pltpu.async_copy(
            tiles.at[k, :, :, pl.ds(0, 128)],
            o_hbm.at[f, :, tb0 + tbl, :, :], wsem.at[k])

    for k in range(CHUNK):
        pltpu.async_copy(tiles.at[k, :, :, pl.ds(0, 128)],
                         o_hbm.at[k % 2, :, tb0, :, :], wsem.at[k])
    for k in range(CHUNK):
        fire(k, k)

    def chunk_body(ci, carry):
        blk0 = ci * CHUNK
        for k in range(CHUNK):
            drain(blk0 + k, k)
            nxt = blk0 + k + CHUNK

            @pl.when(nxt < BLOCKS_PER_W)
            def _():
                fire(nxt, k)
        return carry

    lax.fori_loop(0, NCHUNKS, chunk_body, 0)
    for k in range(CHUNK):
        pltpu.make_async_copy(
            tiles.at[k, :, :, pl.ds(0, 128)],
            o_hbm.at[0, :, 0, :, :], wsem.at[k]).wait()


@jax.jit
def _emb(idx_flat, weight):
    mesh = plsc.VectorSubcoreMesh(core_axis_name="c", subcore_axis_name="s")
    run = pl.kernel(
        _emb_body,
        out_type=jax.ShapeDtypeStruct((FIELDS, 4, NTB, 8, 128), jnp.float32),
        mesh=mesh,
        scratch_types=[
            pltpu.VMEM((PER_W,), jnp.int32),
            pltpu.VMEM((CHUNK, 128), jnp.int32),
            pltpu.VMEM((CHUNK, 128, EMBED), jnp.float32),
            pltpu.VMEM((CHUNK, 4, 8, TPAD), jnp.float32),
            pltpu.SemaphoreType.DMA((CHUNK,)),
            pltpu.SemaphoreType.DMA((CHUNK,)),
        ],
        compiler_params=pltpu.CompilerParams(
            use_tc_tiling_on_sc=False, needs_layout_passes=False),
    )
    return run(idx_flat, weight)


def kernel(input, weight):
    idx_flat = input.reshape(TOTAL).astype(jnp.int32)
    out = _emb(idx_flat, weight)
    return out.transpose(2, 4, 0, 1, 3).reshape(BATCH, FIELDS, EMBED)
